# baseline (speedup 1.0000x reference)
import functools

import jax
import jax.numpy as jnp
from jax import lax
from jax.experimental import pallas as pl
from jax.experimental.pallas import tpu as pltpu
from jax.experimental.pallas import tpu_sc as plsc

_INV_T = 1.0 / 0.07

_NC = 2
_NS = 16
_L = 16
_NW = _NC * _NS

_B = 1024
_D = 128
_KP1 = 256
_V = 100000

_BPW = _B // _NW
_RPW = _V // _NW
_CCH = 256
_TAIL2 = _RPW - 32 * 80 - 2 * _CCH


def _rsqrt(x):
    u = plsc.bitcast(x, jnp.uint32)
    u = jnp.uint32(0x5F3759DF) - (u >> jnp.uint32(1))
    y = plsc.bitcast(u, jnp.float32)
    for _ in range(4):
        y = y * (1.5 - 0.5 * x * y * y)
    return y


def _sc_body(ab_hbm, l_hbm, ss_hbm, idx_hbm, y_hbm, mem_hbm,
             out3_hbm, newmem_hbm,
             idx_v, abv, lv, ssv, rowsA, rowsB, aux0, aux1, aux2, outv,
             cbufA, cbufB,
             yv, owned, auxy, auxi, auxn, auxr, mrows, arows, ubuf,
             semA, semB, semC, semO, semU, semW, semV):
    wid = lax.axis_index("s") * _NC + lax.axis_index("c")
    b0 = pl.multiple_of(wid * _BPW, _BPW)
    lane = lax.broadcasted_iota(jnp.int32, (_L,), 0)

    pltpu.sync_copy(idx_hbm.at[pl.ds(b0, _BPW)], idx_v)
    pltpu.sync_copy(ab_hbm.at[pl.ds(b0, _BPW)], abv)
    pltpu.sync_copy(l_hbm.at[pl.ds(b0, _BPW)], lv)
    pltpu.sync_copy(ss_hbm.at[pl.ds(b0, _BPW)], ssv)

    def start_gather(jj, buf, sem):
        return pltpu.async_copy(mem_hbm.at[idx_v.at[jj]], buf, sem)

    def wait_gather(jj, buf, sem):
        pltpu.make_async_copy(mem_hbm.at[idx_v.at[jj]], buf, sem).wait()

    def compute_b(jj, rows, p):
        vas = [abv[jj, pl.ds(c * _L, _L)] for c in range(_D // _L)]
        vls = [lv[jj, pl.ds(c * _L, _L)] for c in range(_D // _L)]
        vss = [ssv[jj, pl.ds(c * _L, _L)] for c in range(_D // _L)]

        @plsc.parallel_loop(0, _KP1, 1, unroll=5)
        def kbody(k):
            z = jnp.zeros((_L,), jnp.float32)
            a0, a1, a2 = z, z, z
            for c in range(_D // _L):
                r = rows[k, pl.ds(c * _L, _L)]
                a0 = a0 + r * vas[c]
                a1 = a1 + r * vls[c]
                a2 = a2 + r * vss[c]
            off = pl.multiple_of(k * _L, 8)
            aux0[pl.ds(off, _L)] = plsc.cumsum(a0)
            aux1[pl.ds(off, _L)] = plsc.cumsum(a1)
            aux2[pl.ds(off, _L)] = plsc.cumsum(a2)

        for kg in range(_KP1 // _L):
            sel = (lane + kg * _L) * _L + (_L - 1)
            outv[p, 0, pl.ds(kg * _L, _L)] = plsc.load_gather(aux0, [sel]) * _INV_T
            outv[p, 1, pl.ds(kg * _L, _L)] = plsc.load_gather(aux1, [sel]) * _INV_T
            outv[p, 2, pl.ds(kg * _L, _L)] = plsc.load_gather(aux2, [sel]) * _INV_T

    _CPR = 80
    r0 = pl.multiple_of(wid * _RPW, 1)

    def cin(s, buf):
        pltpu.async_copy(mem_hbm.at[pl.ds(r0 + s * _CPR, _CPR)], buf, semC)

    def cin_wait(s, buf):
        pltpu.make_async_copy(mem_hbm.at[pl.ds(r0 + s * _CPR, _CPR)], buf,
                              semC).wait()

    def cout(s, buf):
        pltpu.async_copy(buf, newmem_hbm.at[pl.ds(r0 + s * _CPR, _CPR)],
                         semO)

    def cout_wait(s, buf):
        pltpu.make_async_copy(buf, newmem_hbm.at[pl.ds(r0 + s * _CPR, _CPR)],
                              semO).wait()

    def outv_start(jj, p):
        pltpu.async_copy(outv.at[p], out3_hbm.at[b0 + jj], semV)

    def outv_wait(jj, p):
        pltpu.make_async_copy(outv.at[p], out3_hbm.at[b0 + jj], semV).wait()

    start_gather(0, rowsA, semA)

    def body(t, _):
        j0 = 2 * t
        j1 = 2 * t + 1

        @pl.when(t >= 1)
        def _():
            cout_wait(j0 - 2, cbufA)

        cin(j0, cbufA)

        @pl.when(t >= 1)
        def _():
            cin_wait(j0 - 1, cbufB)
            cout(j0 - 1, cbufB)
            outv_wait(j0 - 2, 0)

        start_gather(j1, rowsB, semB)
        wait_gather(j0, rowsA, semA)
        compute_b(j0, rowsA, 0)
        outv_start(j0, 0)

        @pl.when(t >= 1)
        def _():
            cout_wait(j1 - 2, cbufB)

        cin(j1, cbufB)
        cin_wait(j0, cbufA)
        cout(j0, cbufA)

        @pl.when(t >= 1)
        def _():
            outv_wait(j1 - 2, 1)

        @pl.when(t + 1 < _BPW // 2)
        def _():
            start_gather(j0 + 2, rowsA, semA)

        wait_gather(j1, rowsB, semB)
        compute_b(j1, rowsB, 1)
        outv_start(j1, 1)
        return 0

    lax.fori_loop(0, _BPW // 2, body, 0)

    cin_wait(_BPW - 1, cbufB)
    cout(_BPW - 1, cbufB)
    cout_wait(_BPW - 2, cbufA)
    cout_wait(_BPW - 1, cbufB)
    outv_wait(_BPW - 2, 0)
    outv_wait(_BPW - 1, 1)

    rem0 = r0 + _BPW * _CPR
    for off, cnt in ((0, _CCH), (_CCH, _CCH), (2 * _CCH, _TAIL2)):
        pltpu.sync_copy(mem_hbm.at[pl.ds(rem0 + off, cnt)],
                        rowsA.at[pl.ds(0, cnt)])
        pltpu.sync_copy(rowsA.at[pl.ds(0, cnt)],
                        newmem_hbm.at[pl.ds(rem0 + off, cnt)])

    pltpu.sync_copy(y_hbm, yv)

    def scan_step(t, cursor):
        yvec = yv[pl.ds(pl.multiple_of(t * _L, 8), _L)]
        m = (yvec >= r0) & (yvec < r0 + _RPW)
        ivec = t * _L + lane
        plsc.store_compressed(owned.at[pl.ds(cursor, _L)], ivec, mask=m)
        cnt = plsc.all_reduce_population_count(m)
        cnt_s = cnt if cnt.ndim == 0 else jnp.max(cnt)
        return cursor + cnt_s

    n = lax.fori_loop(0, _B // _L, scan_step, 0)

    @pl.when(n > 0)
    def _():
        last = plsc.load_gather(owned, [jnp.full((_L,), n - 1, jnp.int32)])

        def sc_step(t, _):
            @pl.when(t * _L < n)
            def _():
                ivec0 = owned[pl.ds(pl.multiple_of(t * _L, 8), _L)]
                lm = (t * _L + lane) < n
                ivec = jnp.where(lm, ivec0, last)
                yvals = plsc.load_gather(yv, [ivec])
                auxy[...] = yvals
                auxi[...] = ivec
                ii = ivec
                for s in range(1, _L):
                    idxs = jnp.minimum(lane + s, _L - 1)
                    sy = plsc.load_gather(auxy, [idxs])
                    si = plsc.load_gather(auxi, [idxs])
                    match = (sy == yvals) & ((lane + s) < _L)
                    ii = jnp.where(match, si, ii)
                pltpu.async_copy(mem_hbm.at[yvals], mrows, semU).wait()
                pltpu.async_copy(ab_hbm.at[ii], arows, semU).wait()
                for r in range(_L):
                    acc = jnp.zeros((_L,), jnp.float32)
                    for c in range(_D // _L):
                        mval = mrows[r, pl.ds(c * _L, _L)]
                        aval = arows[r, pl.ds(c * _L, _L)]
                        p = (mval + aval) * 0.5
                        ubuf[r, pl.ds(c * _L, _L)] = p
                        acc = acc + p * p
                    auxn[pl.ds(r * _L, _L)] = plsc.cumsum(acc)
                norms = plsc.load_gather(auxn, [lane * _L + (_L - 1)])
                auxr[...] = _rsqrt(norms)
                for r in range(_L):
                    scl = plsc.load_gather(auxr, [jnp.full((_L,), r,
                                                           jnp.int32)])
                    for c in range(_D // _L):
                        ubuf[r, pl.ds(c * _L, _L)] = (
                            ubuf[r, pl.ds(c * _L, _L)] * scl)
                pltpu.async_copy(ubuf, newmem_hbm.at[yvals], semW).wait()
            return 0

        lax.fori_loop(0, _B // _L, sc_step, 0)


@functools.partial(
    pl.kernel,
    out_type=[
        jax.ShapeDtypeStruct((_B, 3, _KP1), jnp.float32),
        jax.ShapeDtypeStruct((_V, _D), jnp.float32),
    ],
    mesh=plsc.VectorSubcoreMesh(core_axis_name="c", subcore_axis_name="s"),
    compiler_params=pltpu.CompilerParams(
        needs_layout_passes=False, use_tc_tiling_on_sc=False),
    scratch_types=[
        pltpu.VMEM((_BPW, _KP1), jnp.int32),
        pltpu.VMEM((_BPW, _D), jnp.float32),
        pltpu.VMEM((_BPW, _D), jnp.float32),
        pltpu.VMEM((_BPW, _D), jnp.float32),
        pltpu.VMEM((_KP1, _D), jnp.float32),
        pltpu.VMEM((_KP1, _D), jnp.float32),
        pltpu.VMEM((_KP1 * _L,), jnp.float32),
        pltpu.VMEM((_KP1 * _L,), jnp.float32),
        pltpu.VMEM((_KP1 * _L,), jnp.float32),
        pltpu.VMEM((2, 3, _KP1), jnp.float32),
        pltpu.VMEM((80, _D), jnp.float32),
        pltpu.VMEM((80, _D), jnp.float32),
        pltpu.VMEM((_B,), jnp.int32),
        pltpu.VMEM((_B + _L,), jnp.int32),
        pltpu.VMEM((_L,), jnp.int32),
        pltpu.VMEM((_L,), jnp.int32),
        pltpu.VMEM((_L * _L,), jnp.float32),
        pltpu.VMEM((_L,), jnp.float32),
        pltpu.VMEM((_L, _D), jnp.float32),
        pltpu.VMEM((_L, _D), jnp.float32),
        pltpu.VMEM((_L, _D), jnp.float32),
        pltpu.SemaphoreType.DMA,
        pltpu.SemaphoreType.DMA,
        pltpu.SemaphoreType.DMA,
        pltpu.SemaphoreType.DMA,
        pltpu.SemaphoreType.DMA,
        pltpu.SemaphoreType.DMA,
        pltpu.SemaphoreType.DMA,
    ],
)
def _sc_kernel(*args):
    _sc_body(*args)


def kernel(ab, l, ss, y, idx, memory_ab):
    out3, new_memory = _sc_kernel(ab, l, ss, idx, y, memory_ab)
    out_orig = out3[:, 0, :, None]
    out_l = out3[:, 1, :, None]
    out_ss = out3[:, 2, :, None]
    return (out_orig, out_l, out_ss, new_memory)


# speedup vs baseline: 1.0982x; 1.0982x over previous
import functools

import jax
import jax.numpy as jnp
from jax import lax
from jax.experimental import pallas as pl
from jax.experimental.pallas import tpu as pltpu
from jax.experimental.pallas import tpu_sc as plsc

_INV_T = 1.0 / 0.07

_NC = 2
_NS = 16
_L = 16
_NW = _NC * _NS

_B = 1024
_D = 128
_KP1 = 256
_V = 100000

_BPW = _B // _NW
_RPW = _V // _NW
_CCH = 256
_TAIL2 = _RPW - 32 * 80 - 2 * _CCH


def _rsqrt(x):
    u = plsc.bitcast(x, jnp.uint32)
    u = jnp.uint32(0x5F3759DF) - (u >> jnp.uint32(1))
    y = plsc.bitcast(u, jnp.float32)
    for _ in range(4):
        y = y * (1.5 - 0.5 * x * y * y)
    return y


def _sc_body(ab_hbm, l_hbm, ss_hbm, idx_hbm, y_hbm, mem_hbm,
             out3_hbm, newmem_hbm,
             idx_v, abv, lv, ssv, rowsA, rowsB, aux0, aux1, aux2, outv,
             cbufA, cbufB,
             yv, owned, auxy, auxi, auxn, auxr, mrows, arows, ubuf,
             semA, semB, semC, semO, semU, semW, semV):
    wid = lax.axis_index("s") * _NC + lax.axis_index("c")
    b0 = pl.multiple_of(wid * _BPW, _BPW)
    lane = lax.broadcasted_iota(jnp.int32, (_L,), 0)

    pltpu.sync_copy(idx_hbm.at[pl.ds(b0, _BPW)], idx_v)
    pltpu.sync_copy(ab_hbm.at[pl.ds(b0, _BPW)], abv)
    pltpu.sync_copy(l_hbm.at[pl.ds(b0, _BPW)], lv)
    pltpu.sync_copy(ss_hbm.at[pl.ds(b0, _BPW)], ssv)

    def start_gather(jj, buf, sem):
        return pltpu.async_copy(mem_hbm.at[idx_v.at[jj]], buf, sem)

    def wait_gather(jj, buf, sem):
        pltpu.make_async_copy(mem_hbm.at[idx_v.at[jj]], buf, sem).wait()

    def compute_b(jj, rows, p):
        vas = [abv[jj, pl.ds(c * _L, _L)] for c in range(_D // _L)]
        vls = [lv[jj, pl.ds(c * _L, _L)] for c in range(_D // _L)]
        vss = [ssv[jj, pl.ds(c * _L, _L)] for c in range(_D // _L)]

        @plsc.parallel_loop(0, _KP1, 1, unroll=4)
        def kbody(k):
            z = jnp.zeros((_L,), jnp.float32)
            a0, a1, a2 = z, z, z
            for c in range(_D // _L):
                r = rows[k, pl.ds(c * _L, _L)]
                a0 = a0 + r * vas[c]
                a1 = a1 + r * vls[c]
                a2 = a2 + r * vss[c]
            off = pl.multiple_of(k * _L, 8)
            aux0[pl.ds(off, _L)] = plsc.cumsum(a0)
            aux1[pl.ds(off, _L)] = plsc.cumsum(a1)
            aux2[pl.ds(off, _L)] = plsc.cumsum(a2)

        for kg in range(_KP1 // _L):
            sel = (lane + kg * _L) * _L + (_L - 1)
            outv[p, 0, pl.ds(kg * _L, _L)] = plsc.load_gather(aux0, [sel]) * _INV_T
            outv[p, 1, pl.ds(kg * _L, _L)] = plsc.load_gather(aux1, [sel]) * _INV_T
            outv[p, 2, pl.ds(kg * _L, _L)] = plsc.load_gather(aux2, [sel]) * _INV_T

    _CPR = 80
    r0 = pl.multiple_of(wid * _RPW, 1)

    def cin(s, buf):
        pltpu.async_copy(mem_hbm.at[pl.ds(r0 + s * _CPR, _CPR)], buf, semC)

    def cin_wait(s, buf):
        pltpu.make_async_copy(mem_hbm.at[pl.ds(r0 + s * _CPR, _CPR)], buf,
                              semC).wait()

    def cout(s, buf):
        pltpu.async_copy(buf, newmem_hbm.at[pl.ds(r0 + s * _CPR, _CPR)],
                         semO)

    def cout_wait(s, buf):
        pltpu.make_async_copy(buf, newmem_hbm.at[pl.ds(r0 + s * _CPR, _CPR)],
                              semO).wait()

    def outv_start(jj, p):
        pltpu.async_copy(outv.at[p], out3_hbm.at[b0 + jj], semV)

    def outv_wait(jj, p):
        pltpu.make_async_copy(outv.at[p], out3_hbm.at[b0 + jj], semV).wait()

    start_gather(0, rowsA, semA)

    def body(t, _):
        j0 = 2 * t
        j1 = 2 * t + 1

        @pl.when(t >= 1)
        def _():
            cout_wait(j0 - 2, cbufA)

        cin(j0, cbufA)

        @pl.when(t >= 1)
        def _():
            cin_wait(j0 - 1, cbufB)
            cout(j0 - 1, cbufB)
            outv_wait(j0 - 2, 0)

        start_gather(j1, rowsB, semB)
        wait_gather(j0, rowsA, semA)
        compute_b(j0, rowsA, 0)
        outv_start(j0, 0)

        @pl.when(t >= 1)
        def _():
            cout_wait(j1 - 2, cbufB)

        cin(j1, cbufB)
        cin_wait(j0, cbufA)
        cout(j0, cbufA)

        @pl.when(t >= 1)
        def _():
            outv_wait(j1 - 2, 1)

        @pl.when(t + 1 < _BPW // 2)
        def _():
            start_gather(j0 + 2, rowsA, semA)

        wait_gather(j1, rowsB, semB)
        compute_b(j1, rowsB, 1)
        outv_start(j1, 1)
        return 0

    lax.fori_loop(0, _BPW // 2, body, 0)

    cin_wait(_BPW - 1, cbufB)
    cout(_BPW - 1, cbufB)
    cout_wait(_BPW - 2, cbufA)
    cout_wait(_BPW - 1, cbufB)
    outv_wait(_BPW - 2, 0)
    outv_wait(_BPW - 1, 1)

    rem0 = r0 + _BPW * _CPR
    for off, cnt in ((0, _CCH), (_CCH, _CCH), (2 * _CCH, _TAIL2)):
        pltpu.sync_copy(mem_hbm.at[pl.ds(rem0 + off, cnt)],
                        rowsA.at[pl.ds(0, cnt)])
        pltpu.sync_copy(rowsA.at[pl.ds(0, cnt)],
                        newmem_hbm.at[pl.ds(rem0 + off, cnt)])

    pltpu.sync_copy(y_hbm, yv)

    def scan_step(t, cursor):
        yvec = yv[pl.ds(pl.multiple_of(t * _L, 8), _L)]
        m = (yvec >= r0) & (yvec < r0 + _RPW)
        ivec = t * _L + lane
        plsc.store_compressed(owned.at[pl.ds(cursor, _L)], ivec, mask=m)
        cnt = plsc.all_reduce_population_count(m)
        cnt_s = cnt if cnt.ndim == 0 else jnp.max(cnt)
        return cursor + cnt_s

    n = lax.fori_loop(0, _B // _L, scan_step, 0)

    @pl.when(n > 0)
    def _():
        last = plsc.load_gather(owned, [jnp.full((_L,), n - 1, jnp.int32)])

        def sc_step(t, _):
            @pl.when(t * _L < n)
            def _():
                ivec0 = owned[pl.ds(pl.multiple_of(t * _L, 8), _L)]
                lm = (t * _L + lane) < n
                ivec = jnp.where(lm, ivec0, last)
                yvals = plsc.load_gather(yv, [ivec])
                auxy[...] = yvals
                auxi[...] = ivec
                ii = ivec
                for s in range(1, _L):
                    idxs = jnp.minimum(lane + s, _L - 1)
                    sy = plsc.load_gather(auxy, [idxs])
                    si = plsc.load_gather(auxi, [idxs])
                    match = (sy == yvals) & ((lane + s) < _L)
                    ii = jnp.where(match, si, ii)
                pltpu.async_copy(mem_hbm.at[yvals], mrows, semU).wait()
                pltpu.async_copy(ab_hbm.at[ii], arows, semU).wait()
                for r in range(_L):
                    acc = jnp.zeros((_L,), jnp.float32)
                    for c in range(_D // _L):
                        mval = mrows[r, pl.ds(c * _L, _L)]
                        aval = arows[r, pl.ds(c * _L, _L)]
                        p = (mval + aval) * 0.5
                        ubuf[r, pl.ds(c * _L, _L)] = p
                        acc = acc + p * p
                    auxn[pl.ds(r * _L, _L)] = plsc.cumsum(acc)
                norms = plsc.load_gather(auxn, [lane * _L + (_L - 1)])
                auxr[...] = _rsqrt(norms)
                for r in range(_L):
                    scl = plsc.load_gather(auxr, [jnp.full((_L,), r,
                                                           jnp.int32)])
                    for c in range(_D // _L):
                        ubuf[r, pl.ds(c * _L, _L)] = (
                            ubuf[r, pl.ds(c * _L, _L)] * scl)
                pltpu.async_copy(ubuf, newmem_hbm.at[yvals], semW).wait()
            return 0

        lax.fori_loop(0, _B // _L, sc_step, 0)


@functools.partial(
    pl.kernel,
    out_type=[
        jax.ShapeDtypeStruct((_B, 3, _KP1), jnp.float32),
        jax.ShapeDtypeStruct((_V, _D), jnp.float32),
    ],
    mesh=plsc.VectorSubcoreMesh(core_axis_name="c", subcore_axis_name="s"),
    compiler_params=pltpu.CompilerParams(
        needs_layout_passes=False, use_tc_tiling_on_sc=False),
    scratch_types=[
        pltpu.VMEM((_BPW, _KP1), jnp.int32),
        pltpu.VMEM((_BPW, _D), jnp.float32),
        pltpu.VMEM((_BPW, _D), jnp.float32),
        pltpu.VMEM((_BPW, _D), jnp.float32),
        pltpu.VMEM((_KP1, _D), jnp.float32),
        pltpu.VMEM((_KP1, _D), jnp.float32),
        pltpu.VMEM((_KP1 * _L,), jnp.float32),
        pltpu.VMEM((_KP1 * _L,), jnp.float32),
        pltpu.VMEM((_KP1 * _L,), jnp.float32),
        pltpu.VMEM((2, 3, _KP1), jnp.float32),
        pltpu.VMEM((80, _D), jnp.float32),
        pltpu.VMEM((80, _D), jnp.float32),
        pltpu.VMEM((_B,), jnp.int32),
        pltpu.VMEM((_B + _L,), jnp.int32),
        pltpu.VMEM((_L,), jnp.int32),
        pltpu.VMEM((_L,), jnp.int32),
        pltpu.VMEM((_L * _L,), jnp.float32),
        pltpu.VMEM((_L,), jnp.float32),
        pltpu.VMEM((_L, _D), jnp.float32),
        pltpu.VMEM((_L, _D), jnp.float32),
        pltpu.VMEM((_L, _D), jnp.float32),
        pltpu.SemaphoreType.DMA,
        pltpu.SemaphoreType.DMA,
        pltpu.SemaphoreType.DMA,
        pltpu.SemaphoreType.DMA,
        pltpu.SemaphoreType.DMA,
        pltpu.SemaphoreType.DMA,
        pltpu.SemaphoreType.DMA,
    ],
)
def _sc_kernel(*args):
    _sc_body(*args)


def kernel(ab, l, ss, y, idx, memory_ab):
    out3, new_memory = _sc_kernel(ab, l, ss, idx, y, memory_ab)
    out_orig = out3[:, 0, :, None]
    out_l = out3[:, 1, :, None]
    out_ss = out3[:, 2, :, None]
    return (out_orig, out_l, out_ss, new_memory)


# speedup vs baseline: 1.1591x; 1.0554x over previous
import functools

import jax
import jax.numpy as jnp
from jax import lax
from jax.experimental import pallas as pl
from jax.experimental.pallas import tpu as pltpu
from jax.experimental.pallas import tpu_sc as plsc

_INV_T = 1.0 / 0.07

_NC = 2
_NS = 16
_L = 16
_NW = _NC * _NS

_B = 1024
_D = 128
_KP1 = 256
_V = 100000

_BPW = _B // _NW
_RPW = _V // _NW
_CCH = 256
_TAIL2 = _RPW - 32 * 80 - 2 * _CCH


def _rsqrt(x):
    u = plsc.bitcast(x, jnp.uint32)
    u = jnp.uint32(0x5F3759DF) - (u >> jnp.uint32(1))
    y = plsc.bitcast(u, jnp.float32)
    for _ in range(4):
        y = y * (1.5 - 0.5 * x * y * y)
    return y


def _sc_body(ab_hbm, l_hbm, ss_hbm, idx_hbm, y_hbm, mem_hbm,
             oa_hbm, ol_hbm, os_hbm, newmem_hbm,
             idx_v, abv, lv, ssv, rowsA, rowsB, aux0, aux1, aux2, outv,
             cbufA, cbufB,
             yv, owned, auxy, auxi, auxn, auxr, mrows, arows, ubuf,
             semA, semB, semC, semO, semU, semW, semV):
    wid = lax.axis_index("s") * _NC + lax.axis_index("c")
    b0 = pl.multiple_of(wid * _BPW, _BPW)
    lane = lax.broadcasted_iota(jnp.int32, (_L,), 0)

    pltpu.sync_copy(idx_hbm.at[pl.ds(b0, _BPW)], idx_v)
    pltpu.sync_copy(ab_hbm.at[pl.ds(b0, _BPW)], abv)
    pltpu.sync_copy(l_hbm.at[pl.ds(b0, _BPW)], lv)
    pltpu.sync_copy(ss_hbm.at[pl.ds(b0, _BPW)], ssv)

    def start_gather(jj, buf, sem):
        return pltpu.async_copy(mem_hbm.at[idx_v.at[jj]], buf, sem)

    def wait_gather(jj, buf, sem):
        pltpu.make_async_copy(mem_hbm.at[idx_v.at[jj]], buf, sem).wait()

    def compute_b(jj, rows, p):
        vas = [abv[jj, pl.ds(c * _L, _L)] for c in range(_D // _L)]
        vls = [lv[jj, pl.ds(c * _L, _L)] for c in range(_D // _L)]
        vss = [ssv[jj, pl.ds(c * _L, _L)] for c in range(_D // _L)]

        @plsc.parallel_loop(0, _KP1, 1, unroll=4)
        def kbody(k):
            z = jnp.zeros((_L,), jnp.float32)
            a0, a1, a2 = z, z, z
            for c in range(_D // _L):
                r = rows[k, pl.ds(c * _L, _L)]
                a0 = a0 + r * vas[c]
                a1 = a1 + r * vls[c]
                a2 = a2 + r * vss[c]
            off = pl.multiple_of(k * _L, 8)
            aux0[pl.ds(off, _L)] = plsc.cumsum(a0)
            aux1[pl.ds(off, _L)] = plsc.cumsum(a1)
            aux2[pl.ds(off, _L)] = plsc.cumsum(a2)

        for kg in range(_KP1 // _L):
            sel = (lane + kg * _L) * _L + (_L - 1)
            outv[p, 0, pl.ds(kg * _L, _L)] = plsc.load_gather(aux0, [sel]) * _INV_T
            outv[p, 1, pl.ds(kg * _L, _L)] = plsc.load_gather(aux1, [sel]) * _INV_T
            outv[p, 2, pl.ds(kg * _L, _L)] = plsc.load_gather(aux2, [sel]) * _INV_T

    _CPR = 80
    r0 = pl.multiple_of(wid * _RPW, 1)

    def cin(s, buf):
        pltpu.async_copy(mem_hbm.at[pl.ds(r0 + s * _CPR, _CPR)], buf, semC)

    def cin_wait(s, buf):
        pltpu.make_async_copy(mem_hbm.at[pl.ds(r0 + s * _CPR, _CPR)], buf,
                              semC).wait()

    def cout(s, buf):
        pltpu.async_copy(buf, newmem_hbm.at[pl.ds(r0 + s * _CPR, _CPR)],
                         semO)

    def cout_wait(s, buf):
        pltpu.make_async_copy(buf, newmem_hbm.at[pl.ds(r0 + s * _CPR, _CPR)],
                              semO).wait()

    def outv_start(jj, p):
        pltpu.async_copy(outv.at[p, 0], oa_hbm.at[b0 + jj], semV)
        pltpu.async_copy(outv.at[p, 1], ol_hbm.at[b0 + jj], semV)
        pltpu.async_copy(outv.at[p, 2], os_hbm.at[b0 + jj], semV)

    def outv_wait(jj, p):
        pltpu.make_async_copy(outv.at[p, 0], oa_hbm.at[b0 + jj], semV).wait()
        pltpu.make_async_copy(outv.at[p, 1], ol_hbm.at[b0 + jj], semV).wait()
        pltpu.make_async_copy(outv.at[p, 2], os_hbm.at[b0 + jj], semV).wait()

    start_gather(0, rowsA, semA)

    def body(t, _):
        j0 = 2 * t
        j1 = 2 * t + 1

        @pl.when(t >= 1)
        def _():
            cout_wait(j0 - 2, cbufA)

        cin(j0, cbufA)

        @pl.when(t >= 1)
        def _():
            cin_wait(j0 - 1, cbufB)
            cout(j0 - 1, cbufB)
            outv_wait(j0 - 2, 0)

        start_gather(j1, rowsB, semB)
        wait_gather(j0, rowsA, semA)
        compute_b(j0, rowsA, 0)
        outv_start(j0, 0)

        @pl.when(t >= 1)
        def _():
            cout_wait(j1 - 2, cbufB)

        cin(j1, cbufB)
        cin_wait(j0, cbufA)
        cout(j0, cbufA)

        @pl.when(t >= 1)
        def _():
            outv_wait(j1 - 2, 1)

        @pl.when(t + 1 < _BPW // 2)
        def _():
            start_gather(j0 + 2, rowsA, semA)

        wait_gather(j1, rowsB, semB)
        compute_b(j1, rowsB, 1)
        outv_start(j1, 1)
        return 0

    lax.fori_loop(0, _BPW // 2, body, 0)

    cin_wait(_BPW - 1, cbufB)
    cout(_BPW - 1, cbufB)
    cout_wait(_BPW - 2, cbufA)
    cout_wait(_BPW - 1, cbufB)
    outv_wait(_BPW - 2, 0)
    outv_wait(_BPW - 1, 1)

    rem0 = r0 + _BPW * _CPR
    for off, cnt in ((0, _CCH), (_CCH, _CCH), (2 * _CCH, _TAIL2)):
        pltpu.sync_copy(mem_hbm.at[pl.ds(rem0 + off, cnt)],
                        rowsA.at[pl.ds(0, cnt)])
        pltpu.sync_copy(rowsA.at[pl.ds(0, cnt)],
                        newmem_hbm.at[pl.ds(rem0 + off, cnt)])

    pltpu.sync_copy(y_hbm, yv)

    def scan_step(t, cursor):
        yvec = yv[pl.ds(pl.multiple_of(t * _L, 8), _L)]
        m = (yvec >= r0) & (yvec < r0 + _RPW)
        ivec = t * _L + lane
        plsc.store_compressed(owned.at[pl.ds(cursor, _L)], ivec, mask=m)
        cnt = plsc.all_reduce_population_count(m)
        cnt_s = cnt if cnt.ndim == 0 else jnp.max(cnt)
        return cursor + cnt_s

    n = lax.fori_loop(0, _B // _L, scan_step, 0)

    @pl.when(n > 0)
    def _():
        last = plsc.load_gather(owned, [jnp.full((_L,), n - 1, jnp.int32)])

        def sc_step(t, _):
            @pl.when(t * _L < n)
            def _():
                ivec0 = owned[pl.ds(pl.multiple_of(t * _L, 8), _L)]
                lm = (t * _L + lane) < n
                ivec = jnp.where(lm, ivec0, last)
                yvals = plsc.load_gather(yv, [ivec])
                auxy[...] = yvals
                auxi[...] = ivec
                ii = ivec
                for s in range(1, _L):
                    idxs = jnp.minimum(lane + s, _L - 1)
                    sy = plsc.load_gather(auxy, [idxs])
                    si = plsc.load_gather(auxi, [idxs])
                    match = (sy == yvals) & ((lane + s) < _L)
                    ii = jnp.where(match, si, ii)
                pltpu.async_copy(mem_hbm.at[yvals], mrows, semU).wait()
                pltpu.async_copy(ab_hbm.at[ii], arows, semU).wait()
                for r in range(_L):
                    acc = jnp.zeros((_L,), jnp.float32)
                    for c in range(_D // _L):
                        mval = mrows[r, pl.ds(c * _L, _L)]
                        aval = arows[r, pl.ds(c * _L, _L)]
                        p = (mval + aval) * 0.5
                        ubuf[r, pl.ds(c * _L, _L)] = p
                        acc = acc + p * p
                    auxn[pl.ds(r * _L, _L)] = plsc.cumsum(acc)
                norms = plsc.load_gather(auxn, [lane * _L + (_L - 1)])
                auxr[...] = _rsqrt(norms)
                for r in range(_L):
                    scl = plsc.load_gather(auxr, [jnp.full((_L,), r,
                                                           jnp.int32)])
                    for c in range(_D // _L):
                        ubuf[r, pl.ds(c * _L, _L)] = (
                            ubuf[r, pl.ds(c * _L, _L)] * scl)
                pltpu.async_copy(ubuf, newmem_hbm.at[yvals], semW).wait()
            return 0

        lax.fori_loop(0, _B // _L, sc_step, 0)


@functools.partial(
    pl.kernel,
    out_type=[
        jax.ShapeDtypeStruct((_B, _KP1), jnp.float32),
        jax.ShapeDtypeStruct((_B, _KP1), jnp.float32),
        jax.ShapeDtypeStruct((_B, _KP1), jnp.float32),
        jax.ShapeDtypeStruct((_V, _D), jnp.float32),
    ],
    mesh=plsc.VectorSubcoreMesh(core_axis_name="c", subcore_axis_name="s"),
    compiler_params=pltpu.CompilerParams(
        needs_layout_passes=False, use_tc_tiling_on_sc=False),
    scratch_types=[
        pltpu.VMEM((_BPW, _KP1), jnp.int32),
        pltpu.VMEM((_BPW, _D), jnp.float32),
        pltpu.VMEM((_BPW, _D), jnp.float32),
        pltpu.VMEM((_BPW, _D), jnp.float32),
        pltpu.VMEM((_KP1, _D), jnp.float32),
        pltpu.VMEM((_KP1, _D), jnp.float32),
        pltpu.VMEM((_KP1 * _L,), jnp.float32),
        pltpu.VMEM((_KP1 * _L,), jnp.float32),
        pltpu.VMEM((_KP1 * _L,), jnp.float32),
        pltpu.VMEM((2, 3, _KP1), jnp.float32),
        pltpu.VMEM((80, _D), jnp.float32),
        pltpu.VMEM((80, _D), jnp.float32),
        pltpu.VMEM((_B,), jnp.int32),
        pltpu.VMEM((_B + _L,), jnp.int32),
        pltpu.VMEM((_L,), jnp.int32),
        pltpu.VMEM((_L,), jnp.int32),
        pltpu.VMEM((_L * _L,), jnp.float32),
        pltpu.VMEM((_L,), jnp.float32),
        pltpu.VMEM((_L, _D), jnp.float32),
        pltpu.VMEM((_L, _D), jnp.float32),
        pltpu.VMEM((_L, _D), jnp.float32),
        pltpu.SemaphoreType.DMA,
        pltpu.SemaphoreType.DMA,
        pltpu.SemaphoreType.DMA,
        pltpu.SemaphoreType.DMA,
        pltpu.SemaphoreType.DMA,
        pltpu.SemaphoreType.DMA,
        pltpu.SemaphoreType.DMA,
    ],
)
def _sc_kernel(*args):
    _sc_body(*args)


def kernel(ab, l, ss, y, idx, memory_ab):
    oa, ol, os, new_memory = _sc_kernel(ab, l, ss, idx, y, memory_ab)
    return (oa[:, :, None], ol[:, :, None], os[:, :, None], new_memory)
